# baseline (speedup 1.0000x reference)
import jax
import jax.numpy as jnp
from jax.experimental import pallas as pl
from jax.experimental.pallas import tpu as pltpu

CHANNEL_IN = 256
CHANNEL_OUT = 32768
GROUP = 8
BATCH = 128

CB = 8192
NCHUNK = CHANNEL_OUT // CB
NBUF = 2
NOBUF = 2


def _store_grouped_argmax_mask(yt, obuf, oslot):
    y3 = yt.reshape(CB // GROUP, GROUP, BATCH)
    v = y3
    for k in (1, 2, 4):
        v = jnp.maximum(v, pltpu.roll(v, GROUP - k, 1))
    eqf = (y3 == v).astype(jnp.float32)
    obuf[oslot] = eqf.reshape(CB, BATCH).T
    total = jnp.sum(eqf)

    @pl.when(total > float(CB // GROUP * BATCH))
    def _exact_tie_break():
        s = jax.lax.broadcasted_iota(
            jnp.int32, (CB // GROUP, GROUP, BATCH), 1).astype(jnp.float32)
        c = jnp.where(y3 == v, s, jnp.float32(GROUP))
        for k in (1, 2, 4):
            c = jnp.minimum(c, pltpu.roll(c, GROUP - k, 1))
        obuf[oslot] = (s == c).astype(jnp.float32).reshape(CB, BATCH).T


def _pipelined_kernel(x_ref, w_hbm, o_hbm, wbuf, obuf, insem, outsem):
    x = x_ref[...]

    def in_copy(j, slot):
        return pltpu.make_async_copy(
            w_hbm.at[:, pl.ds(j * CB, CB)], wbuf.at[slot], insem.at[slot])

    def out_copy(j, slot):
        return pltpu.make_async_copy(
            obuf.at[slot], o_hbm.at[:, pl.ds(j * CB, CB)], outsem.at[slot])

    for j in range(NBUF):
        in_copy(j, j).start()
    for j in range(NCHUNK):
        slot = j % NBUF
        in_copy(j, slot).wait()
        yt = jax.lax.dot_general(
            wbuf[slot], x, (((0,), (1,)), ((), ())),
            preferred_element_type=jnp.float32)
        oslot = j % NOBUF
        if j >= NOBUF:
            out_copy(j - NOBUF, oslot).wait()
        _store_grouped_argmax_mask(yt, obuf, oslot)
        out_copy(j, oslot).start()
        nxt = j + NBUF
        if nxt < NCHUNK:
            in_copy(nxt, slot).start()
    for j in range(NCHUNK - NOBUF, NCHUNK):
        out_copy(j, j % NOBUF).wait()


def kernel(x, W):
    return pl.pallas_call(
        _pipelined_kernel,
        in_specs=[
            pl.BlockSpec(memory_space=pltpu.VMEM),
            pl.BlockSpec(memory_space=pl.ANY),
        ],
        out_specs=pl.BlockSpec(memory_space=pl.ANY),
        out_shape=jax.ShapeDtypeStruct((BATCH, CHANNEL_OUT), jnp.float32),
        scratch_shapes=[
            pltpu.VMEM((NBUF, CHANNEL_IN, CB), jnp.float32),
            pltpu.VMEM((NOBUF, BATCH, CB), jnp.float32),
            pltpu.SemaphoreType.DMA((NBUF,)),
            pltpu.SemaphoreType.DMA((NOBUF,)),
        ],
    )(x, W)


# speedup vs baseline: 1.1423x; 1.1423x over previous
import jax
import jax.numpy as jnp
from jax.experimental import pallas as pl
from jax.experimental.pallas import tpu as pltpu

CHANNEL_IN = 256
CHANNEL_OUT = 32768
GROUP = 8
BATCH = 128

COL_BLK = 8192


def _fused_kernel(x_ref, w_ref, o_ref):
    yt = jax.lax.dot_general(
        w_ref[...], x_ref[...], (((0,), (1,)), ((), ())),
        preferred_element_type=jnp.float32)
    y3 = yt.reshape(COL_BLK // GROUP, GROUP, BATCH)
    v = y3
    for k in (1, 2, 4):
        v = jnp.maximum(v, pltpu.roll(v, GROUP - k, 1))
    eqf = (y3 == v).astype(jnp.float32)
    o_ref[...] = eqf.reshape(COL_BLK, BATCH).T
    total = jnp.sum(eqf)

    @pl.when(total > float(COL_BLK // GROUP * BATCH))
    def _exact_tie_break():
        s = jax.lax.broadcasted_iota(
            jnp.int32, (COL_BLK // GROUP, GROUP, BATCH), 1).astype(jnp.float32)
        c = jnp.where(y3 == v, s, jnp.float32(GROUP))
        for k in (1, 2, 4):
            c = jnp.minimum(c, pltpu.roll(c, GROUP - k, 1))
        o_ref[...] = (s == c).astype(jnp.float32).reshape(COL_BLK, BATCH).T


def kernel(x, W):
    grid = (CHANNEL_OUT // COL_BLK,)
    return pl.pallas_call(
        _fused_kernel,
        grid=grid,
        in_specs=[
            pl.BlockSpec((BATCH, CHANNEL_IN), lambda j: (0, 0)),
            pl.BlockSpec((CHANNEL_IN, COL_BLK), lambda j: (0, j)),
        ],
        out_specs=pl.BlockSpec((BATCH, COL_BLK), lambda j: (0, j)),
        out_shape=jax.ShapeDtypeStruct((BATCH, CHANNEL_OUT), jnp.float32),
        compiler_params=pltpu.CompilerParams(
            dimension_semantics=("parallel",),
        ),
    )(x, W)
